# Initial kernel scaffold; baseline (speedup 1.0000x reference)
import jax
import jax.numpy as jnp
from jax.experimental import pallas as pl


def kernel(x, lut):
    raise NotImplementedError("write your pallas kernel here")


# speedup vs baseline: 279.4780x; 279.4780x over previous
import functools

import jax
import jax.numpy as jnp
from jax import lax
from jax.experimental import pallas as pl
from jax.experimental.pallas import tpu as pltpu
from jax.experimental.pallas import tpu_sc as plsc

_NC = 2
_NS = 16
_NW = _NC * _NS
_L = 16

_TOTAL = 3072 * 4096
_PER_W = _TOTAL // _NW
_CH = 16384
_NCH = _PER_W // _CH
_LUT_N = 4096


def _sc_body(x_hbm, lut_hbm, out_hbm, lut_v, x_v, o_v):
    wid = lax.axis_index("s") * _NC + lax.axis_index("c")
    base = wid * _PER_W
    pltpu.sync_copy(lut_hbm, lut_v)

    def chunk_body(c, carry):
        off = base + c * _CH
        pltpu.sync_copy(x_hbm.at[pl.ds(off, _CH)], x_v)

        def gather_step(i, carry2):
            idx = x_v[pl.ds(i * _L, _L)]
            idx = jnp.minimum(jnp.maximum(idx, 0), _LUT_N - 1)
            o_v[pl.ds(i * _L, _L)] = plsc.load_gather(lut_v, [idx])
            return carry2

        lax.fori_loop(0, _CH // _L, gather_step, 0, unroll=8)
        pltpu.sync_copy(o_v, out_hbm.at[pl.ds(off, _CH)])
        return carry

    lax.fori_loop(0, _NCH, chunk_body, 0)


_sc_kernel = functools.partial(
    pl.kernel,
    mesh=plsc.VectorSubcoreMesh(core_axis_name="c", subcore_axis_name="s"),
    out_type=jax.ShapeDtypeStruct((_TOTAL,), jnp.float32),
    scratch_types=[
        pltpu.VMEM((_LUT_N,), jnp.float32),
        pltpu.VMEM((_CH,), jnp.int32),
        pltpu.VMEM((_CH,), jnp.float32),
    ],
    compiler_params=pltpu.CompilerParams(needs_layout_passes=False),
)(_sc_body)


def kernel(x, lut):
    out_flat = _sc_kernel(x.reshape(-1), lut)
    return out_flat.reshape(x.shape)


# speedup vs baseline: 312.2518x; 1.1173x over previous
import functools

import jax
import jax.numpy as jnp
from jax import lax
from jax.experimental import pallas as pl
from jax.experimental.pallas import tpu as pltpu
from jax.experimental.pallas import tpu_sc as plsc

_NC = 2
_NS = 16
_NW = _NC * _NS
_L = 16

_TOTAL = 3072 * 4096
_PER_W = _TOTAL // _NW
_CH = 16384
_NCH = _PER_W // _CH
_LUT_N = 4096


def _sc_body(x_hbm, lut_hbm, out_hbm, lut_v, x_v0, x_v1, o_v0, o_v1,
             in_s0, in_s1, out_s0, out_s1):
    wid = lax.axis_index("s") * _NC + lax.axis_index("c")
    base = wid * _PER_W
    x_bufs = (x_v0, x_v1)
    o_bufs = (o_v0, o_v1)
    in_sems = (in_s0, in_s1)
    out_sems = (out_s0, out_s1)

    pltpu.sync_copy(lut_hbm, lut_v)

    def start_in(c, b):
        pltpu.async_copy(x_hbm.at[pl.ds(base + c * _CH, _CH)], x_bufs[b],
                         in_sems[b])

    def start_out(c, b):
        pltpu.async_copy(o_bufs[b], out_hbm.at[pl.ds(base + c * _CH, _CH)],
                         out_sems[b])

    def drain(src, dst, sem):
        pltpu.make_async_copy(src, dst, sem).wait()

    start_in(0, 0)
    start_in(1, 1)

    @pl.loop(0, _NCH, step=2)
    def chunk_body(c0):
        for b in range(2):
            c = c0 + b
            drain(x_hbm.at[pl.ds(0, _CH)], x_bufs[b], in_sems[b])

            @pl.when(c >= 2)
            def _():
                drain(o_bufs[b], out_hbm.at[pl.ds(0, _CH)], out_sems[b])

            def gather_step(i, carry2):
                idx = x_bufs[b][pl.ds(i * _L, _L)]
                idx = jnp.minimum(jnp.maximum(idx, 0), _LUT_N - 1)
                o_bufs[b][pl.ds(i * _L, _L)] = plsc.load_gather(lut_v, [idx])
                return carry2

            lax.fori_loop(0, _CH // _L, gather_step, 0, unroll=8)
            start_out(c, b)

            @pl.when(c + 2 < _NCH)
            def _():
                start_in(c + 2, b)
    drain(o_bufs[0], out_hbm.at[pl.ds(0, _CH)], out_sems[0])
    drain(o_bufs[1], out_hbm.at[pl.ds(0, _CH)], out_sems[1])


_sc_kernel = functools.partial(
    pl.kernel,
    mesh=plsc.VectorSubcoreMesh(core_axis_name="c", subcore_axis_name="s"),
    out_type=jax.ShapeDtypeStruct((_TOTAL,), jnp.float32),
    scratch_types=[
        pltpu.VMEM((_LUT_N,), jnp.float32),
        pltpu.VMEM((_CH,), jnp.int32),
        pltpu.VMEM((_CH,), jnp.int32),
        pltpu.VMEM((_CH,), jnp.float32),
        pltpu.VMEM((_CH,), jnp.float32),
        pltpu.SemaphoreType.DMA,
        pltpu.SemaphoreType.DMA,
        pltpu.SemaphoreType.DMA,
        pltpu.SemaphoreType.DMA,
    ],
    compiler_params=pltpu.CompilerParams(needs_layout_passes=False),
)(_sc_body)


def kernel(x, lut):
    out_flat = _sc_kernel(x.reshape(-1), lut)
    return out_flat.reshape(x.shape)


# speedup vs baseline: 791.2812x; 2.5341x over previous
import functools

import jax
import jax.numpy as jnp
from jax import lax
from jax.experimental import pallas as pl
from jax.experimental.pallas import tpu as pltpu
from jax.experimental.pallas import tpu_sc as plsc

_NC = 2
_NS = 16
_NW = _NC * _NS
_L = 16

_TOTAL = 3072 * 4096
_PER_W = _TOTAL // _NW
_CH = 16384
_NCH = _PER_W // _CH
_LUT_N = 4096


def _sc_body(x_hbm, lut_hbm, out_hbm, lut_v, x_v0, x_v1, o_v0, o_v1,
             in_s0, in_s1, out_s0, out_s1):
    wid = lax.axis_index("s") * _NC + lax.axis_index("c")
    base = wid * _PER_W
    x_bufs = (x_v0, x_v1)
    o_bufs = (o_v0, o_v1)
    in_sems = (in_s0, in_s1)
    out_sems = (out_s0, out_s1)

    pltpu.sync_copy(lut_hbm, lut_v)

    def start_in(c, b):
        pltpu.async_copy(x_hbm.at[pl.ds(base + c * _CH, _CH)], x_bufs[b],
                         in_sems[b])

    def start_out(c, b):
        pltpu.async_copy(o_bufs[b], out_hbm.at[pl.ds(base + c * _CH, _CH)],
                         out_sems[b])

    def drain(src, dst, sem):
        pltpu.make_async_copy(src, dst, sem).wait()

    start_in(0, 0)
    start_in(1, 1)

    @pl.loop(0, _NCH, step=2)
    def chunk_body(c0):
        for b in range(2):
            c = c0 + b
            drain(x_hbm.at[pl.ds(0, _CH)], x_bufs[b], in_sems[b])

            @pl.when(c >= 2)
            def _():
                drain(o_bufs[b], out_hbm.at[pl.ds(0, _CH)], out_sems[b])

            @plsc.parallel_loop(0, _CH, step=_L, unroll=8)
            def gather_step(i):
                idx = x_bufs[b][pl.ds(i, _L)]
                idx = jnp.minimum(jnp.maximum(idx, 0), _LUT_N - 1)
                o_bufs[b][pl.ds(i, _L)] = plsc.load_gather(lut_v, [idx])

            start_out(c, b)

            @pl.when(c + 2 < _NCH)
            def _():
                start_in(c + 2, b)
    drain(o_bufs[0], out_hbm.at[pl.ds(0, _CH)], out_sems[0])
    drain(o_bufs[1], out_hbm.at[pl.ds(0, _CH)], out_sems[1])


_sc_kernel = functools.partial(
    pl.kernel,
    mesh=plsc.VectorSubcoreMesh(core_axis_name="c", subcore_axis_name="s"),
    out_type=jax.ShapeDtypeStruct((_TOTAL,), jnp.float32),
    scratch_types=[
        pltpu.VMEM((_LUT_N,), jnp.float32),
        pltpu.VMEM((_CH,), jnp.int32),
        pltpu.VMEM((_CH,), jnp.int32),
        pltpu.VMEM((_CH,), jnp.float32),
        pltpu.VMEM((_CH,), jnp.float32),
        pltpu.SemaphoreType.DMA,
        pltpu.SemaphoreType.DMA,
        pltpu.SemaphoreType.DMA,
        pltpu.SemaphoreType.DMA,
    ],
    compiler_params=pltpu.CompilerParams(needs_layout_passes=False),
)(_sc_body)


def kernel(x, lut):
    out_flat = _sc_kernel(x.reshape(-1), lut)
    return out_flat.reshape(x.shape)


# speedup vs baseline: 1881.8160x; 2.3782x over previous
import functools

import jax
import jax.numpy as jnp
from jax import lax
from jax.experimental import pallas as pl
from jax.experimental.pallas import tpu as pltpu
from jax.experimental.pallas import tpu_sc as plsc

_NC = 2
_NS = 16
_NW = _NC * _NS
_L = 16

_ROWS = 3072
_COLS = 4096
_ROWS_W = _ROWS // _NW
_SR = 8
_SC_COLS = 2048
_NCOL = _COLS // _SC_COLS
_NCH = (_ROWS_W // _SR) * _NCOL
_LUT_N = 4096


def _sc_body(x_hbm, lut_hbm, out_hbm, lut_v, x_v0, x_v1, o_v0, o_v1,
             in_s0, in_s1, out_s0, out_s1):
    wid = lax.axis_index("s") * _NC + lax.axis_index("c")
    base_row = wid * _ROWS_W
    x_bufs = (x_v0, x_v1)
    o_bufs = (o_v0, o_v1)
    in_sems = (in_s0, in_s1)
    out_sems = (out_s0, out_s1)

    pltpu.sync_copy(lut_hbm, lut_v)

    def slab(c):
        r0 = base_row + (c // _NCOL) * _SR
        c0 = (c % _NCOL) * _SC_COLS
        return r0, c0

    def start_in(c, b):
        r0, c0 = slab(c)
        pltpu.async_copy(x_hbm.at[pl.ds(r0, _SR), pl.ds(c0, _SC_COLS)],
                         x_bufs[b], in_sems[b])

    def start_out(c, b):
        r0, c0 = slab(c)
        pltpu.async_copy(o_bufs[b],
                         out_hbm.at[pl.ds(r0, _SR), pl.ds(c0, _SC_COLS)],
                         out_sems[b])

    def drain_in(b):
        pltpu.make_async_copy(x_hbm.at[pl.ds(0, _SR), pl.ds(0, _SC_COLS)],
                              x_bufs[b], in_sems[b]).wait()

    def drain_out(b):
        pltpu.make_async_copy(o_bufs[b],
                              out_hbm.at[pl.ds(0, _SR), pl.ds(0, _SC_COLS)],
                              out_sems[b]).wait()

    start_in(0, 0)
    start_in(1, 1)

    @pl.loop(0, _NCH, step=2)
    def chunk_body(c0):
        for b in range(2):
            c = c0 + b
            drain_in(b)

            @pl.when(c >= 2)
            def _():
                drain_out(b)

            @plsc.parallel_loop(0, _SC_COLS, step=_L, unroll=2)
            def gather_step(i):
                for r in range(_SR):
                    idx = x_bufs[b][r, pl.ds(i, _L)]
                    idx = jnp.minimum(jnp.maximum(idx, 0), _LUT_N - 1)
                    o_bufs[b][r, pl.ds(i, _L)] = plsc.load_gather(
                        lut_v, [idx])

            start_out(c, b)

            @pl.when(c + 2 < _NCH)
            def _():
                start_in(c + 2, b)

    drain_out(0)
    drain_out(1)


_sc_kernel = functools.partial(
    pl.kernel,
    mesh=plsc.VectorSubcoreMesh(core_axis_name="c", subcore_axis_name="s"),
    out_type=jax.ShapeDtypeStruct((_ROWS, _COLS), jnp.float32),
    scratch_types=[
        pltpu.VMEM((_LUT_N,), jnp.float32),
        pltpu.VMEM((_SR, _SC_COLS), jnp.int32),
        pltpu.VMEM((_SR, _SC_COLS), jnp.int32),
        pltpu.VMEM((_SR, _SC_COLS), jnp.float32),
        pltpu.VMEM((_SR, _SC_COLS), jnp.float32),
        pltpu.SemaphoreType.DMA,
        pltpu.SemaphoreType.DMA,
        pltpu.SemaphoreType.DMA,
        pltpu.SemaphoreType.DMA,
    ],
    compiler_params=pltpu.CompilerParams(needs_layout_passes=False),
)(_sc_body)


def kernel(x, lut):
    return _sc_kernel(x, lut)
